# baseline (speedup 1.0000x reference)
import jax
import jax.numpy as jnp
from jax import lax
from jax.experimental import pallas as pl
from jax.experimental.pallas import tpu as pltpu
from jax.experimental.pallas import tpu_sc as plsc

N_NODES = 10000
N_EDGES = 320000
H_DIM = 128
NUM_RELS = 8

NC = 2
NS = 16
L = 16
NW = NC * NS
EPW = N_EDGES // NW
B = 80
NB = EPW // B
NG = B // L
W_DIM = H_DIM // 2
NC2 = W_DIM // L
NPAIR = (NB - 1) // 2

_ILV = plsc.PackFormat.INTERLEAVED


def _sc_body(idx_hbm, w_hbm, h_hbm, out_hbm,
             i0, i1, rc0, rc1, sr0, dr0, sr1, dr1,
             wv, colbuf, score, sem0, sem1):
    wid = lax.axis_index("s") * NC + lax.axis_index("c")
    bid0 = wid * NB
    lane = lax.iota(jnp.int32, L)

    pltpu.sync_copy(w_hbm, wv)

    def issue(i_ref, sr, dr, sem):
        pltpu.async_copy(h_hbm.at[i_ref.at[0]], sr, sem)
        pltpu.async_copy(h_hbm.at[i_ref.at[1]], dr, sem)

    def drain(i_ref, rc, sr, dr, sem):
        pltpu.make_async_copy(h_hbm.at[i_ref.at[0]], sr, sem).wait()
        pltpu.make_async_copy(h_hbm.at[i_ref.at[1]], dr, sem).wait()
        for k in range(NG):
            rc[pl.ds(k * L, L)] = i_ref[2, pl.ds(k * L, L)]

    def fma_edge(e, sr, dr, wrow, acc):
        for c2 in range(NC2):
            sb = plsc.bitcast(sr[e, pl.ds(c2 * L, L)], jnp.bfloat16)
            tb = plsc.bitcast(dr[e, pl.ds(c2 * L, L)], jnp.bfloat16)
            se, so = plsc.unpack(sb, format=_ILV)
            te, to = plsc.unpack(tb, format=_ILV)
            acc = acc + se * te * wrow[2 * c2]
            acc = acc + so * to * wrow[2 * c2 + 1]
        return acc

    def compute(bofs, rc, sr, dr):
        def group_body(g, carry):
            e0 = g * L
            rvg = rc[pl.ds(e0, L)]
            rsp0 = plsc.load_gather(rc, [jnp.full((L,), e0, jnp.int32)])
            nmix = jnp.sum(jnp.where(rvg != rsp0, 1, 0))

            @pl.when(nmix == 0)
            def _fast():
                wrow = [plsc.load_gather(wv, [rsp0 * H_DIM + c * L + lane])
                        for c in range(2 * NC2)]
                for j in range(L):
                    e = e0 + j
                    acc = fma_edge(e, sr, dr, wrow,
                                   jnp.zeros((L,), jnp.float32))
                    colbuf[pl.ds(j * L, L)] = acc

            @pl.when(nmix != 0)
            def _slow():
                for j in range(L):
                    e = e0 + j
                    rsp = plsc.load_gather(rc, [jnp.full((L,), e, jnp.int32)])
                    wrow = [plsc.load_gather(wv, [rsp * H_DIM + c * L + lane])
                            for c in range(2 * NC2)]
                    acc = fma_edge(e, sr, dr, wrow,
                                   jnp.zeros((L,), jnp.float32))
                    colbuf[pl.ds(j * L, L)] = acc

            sc = jnp.zeros((L,), jnp.float32)
            for i in range(L):
                sc = sc + plsc.load_gather(colbuf, [lane * L + i])
            score[pl.ds(bofs * B + g * L, L)] = sc
            return carry

        lax.fori_loop(0, NG, group_body, 0)

    pltpu.sync_copy(idx_hbm.at[bid0], i0)
    issue(i0, sr0, dr0, sem0)
    pltpu.sync_copy(idx_hbm.at[bid0 + 1], i1)

    def pair_body(p, carry):
        b0 = 2 * p
        issue(i1, sr1, dr1, sem1)
        drain(i0, rc0, sr0, dr0, sem0)
        pltpu.sync_copy(idx_hbm.at[bid0 + b0 + 2], i0)
        compute(b0, rc0, sr0, dr0)
        issue(i0, sr0, dr0, sem0)
        drain(i1, rc1, sr1, dr1, sem1)

        @pl.when(b0 + 3 < NB)
        def _():
            pltpu.sync_copy(idx_hbm.at[bid0 + b0 + 3], i1)

        compute(b0 + 1, rc1, sr1, dr1)
        return carry

    lax.fori_loop(0, NPAIR, pair_body, 0)

    drain(i0, rc0, sr0, dr0, sem0)
    compute(NB - 1, rc0, sr0, dr0)

    pltpu.sync_copy(score, out_hbm.at[pl.ds(wid * EPW, EPW)])


def kernel(h, edge_index, rel_ids, w_relation):
    u = edge_index[0].astype(jnp.int32)
    v = edge_index[1].astype(jnp.int32)
    r = rel_ids.astype(jnp.int32)
    idx3 = (jnp.stack([u, v, r], axis=0)
            .reshape(3, NW * NB, B)
            .transpose(1, 0, 2))
    h_packed = jax.lax.bitcast_convert_type(
        h.astype(jnp.bfloat16).reshape(N_NODES, W_DIM, 2), jnp.int32)
    w_perm = (w_relation.astype(jnp.float32)
              .reshape(NUM_RELS, NC2, L, 2)
              .transpose(0, 1, 3, 2)
              .reshape(-1))
    run = pl.kernel(
        _sc_body,
        mesh=plsc.VectorSubcoreMesh(core_axis_name="c", subcore_axis_name="s"),
        compiler_params=pltpu.CompilerParams(needs_layout_passes=False,
                                             use_tc_tiling_on_sc=False),
        out_type=jax.ShapeDtypeStruct((N_EDGES,), jnp.float32),
        scratch_types=[
            pltpu.VMEM((3, B), jnp.int32),
            pltpu.VMEM((3, B), jnp.int32),
            pltpu.VMEM((B,), jnp.int32),
            pltpu.VMEM((B,), jnp.int32),
            pltpu.VMEM((B, W_DIM), jnp.int32),
            pltpu.VMEM((B, W_DIM), jnp.int32),
            pltpu.VMEM((B, W_DIM), jnp.int32),
            pltpu.VMEM((B, W_DIM), jnp.int32),
            pltpu.VMEM((NUM_RELS * H_DIM,), jnp.float32),
            pltpu.VMEM((L * L,), jnp.float32),
            pltpu.VMEM((EPW,), jnp.float32),
            pltpu.SemaphoreType.DMA,
            pltpu.SemaphoreType.DMA,
        ],
    )
    return run(idx3, w_perm, h_packed)


# speedup vs baseline: 1.4860x; 1.4860x over previous
import jax
import jax.numpy as jnp
from jax import lax
from jax.experimental import pallas as pl
from jax.experimental.pallas import tpu as pltpu
from jax.experimental.pallas import tpu_sc as plsc

N_NODES = 10000
N_EDGES = 320000
H_DIM = 128
NUM_RELS = 8

NC = 2
NS = 16
L = 16
NW = NC * NS
EPW = N_EDGES // NW
B = 80
NB = EPW // B
NG = B // L
NCHUNK = H_DIM // L
NPAIR = (NB - 1) // 2


def _sc_body(idx_hbm, w_hbm, h_hbm, out_hbm,
             i0, i1, rc0, rc1, sr0, dr0, sr1, dr1,
             wv, colbuf, score, sem0, sem1, semi0, semi1):
    wid = lax.axis_index("s") * NC + lax.axis_index("c")
    bid0 = wid * NB
    lane = lax.iota(jnp.int32, L)

    pltpu.sync_copy(w_hbm, wv)

    def issue(i_ref, sr, dr, sem):
        pltpu.async_copy(h_hbm.at[i_ref.at[0]], sr, sem)
        pltpu.async_copy(h_hbm.at[i_ref.at[1]], dr, sem)

    def drain(i_ref, rc, sr, dr, sem):
        pltpu.make_async_copy(h_hbm.at[i_ref.at[0]], sr, sem).wait()
        pltpu.make_async_copy(h_hbm.at[i_ref.at[1]], dr, sem).wait()
        for k in range(NG):
            rc[pl.ds(k * L, L)] = i_ref[2, pl.ds(k * L, L)]

    def fma_edge(e, sr, dr, wrow):
        acc = jnp.zeros((L,), jnp.float32)
        for c in range(NCHUNK):
            s = sr[e, pl.ds(c * L, L)]
            t = dr[e, pl.ds(c * L, L)]
            acc = acc + s * t * wrow[c]
        return acc

    def wrow_for(rsp):
        return [plsc.load_gather(wv, [rsp * H_DIM + c * L + lane])
                for c in range(NCHUNK)]

    def transpose_store(bofs, g):
        sc = jnp.zeros((L,), jnp.float32)
        for i in range(L):
            sc = sc + plsc.load_gather(colbuf, [lane * L + i])
        score[pl.ds(bofs * B + g * L, L)] = sc

    def compute(bofs, rc, sr, dr):
        rsplat = plsc.load_gather(rc, [jnp.zeros((L,), jnp.int32)])
        cnt = jnp.zeros((L,), jnp.int32)
        for k in range(NG):
            cnt = cnt + jnp.where(rc[pl.ds(k * L, L)] != rsplat, 1, 0)
        nmix_b = jnp.sum(cnt)

        @pl.when(nmix_b == 0)
        def _uniform_batch():
            wrow = wrow_for(rsplat)

            def gb(g, carry):
                for j in range(L):
                    colbuf[pl.ds(j * L, L)] = fma_edge(g * L + j, sr, dr, wrow)
                transpose_store(bofs, g)
                return carry

            lax.fori_loop(0, NG, gb, 0)

        @pl.when(nmix_b != 0)
        def _mixed_batch():
            def gb(g, carry):
                e0 = g * L
                rsp0 = plsc.load_gather(rc, [jnp.full((L,), e0, jnp.int32)])
                nmix_g = jnp.sum(jnp.where(rc[pl.ds(e0, L)] != rsp0, 1, 0))

                @pl.when(nmix_g == 0)
                def _fast():
                    wrow = wrow_for(rsp0)
                    for j in range(L):
                        colbuf[pl.ds(j * L, L)] = fma_edge(e0 + j, sr, dr, wrow)

                @pl.when(nmix_g != 0)
                def _slow():
                    for j in range(L):
                        e = e0 + j
                        rsp = plsc.load_gather(
                            rc, [jnp.full((L,), e, jnp.int32)])
                        colbuf[pl.ds(j * L, L)] = fma_edge(
                            e, sr, dr, wrow_for(rsp))

                transpose_store(bofs, g)
                return carry

            lax.fori_loop(0, NG, gb, 0)

    pltpu.sync_copy(idx_hbm.at[bid0], i0)
    issue(i0, sr0, dr0, sem0)
    pltpu.async_copy(idx_hbm.at[bid0 + 1], i1, semi1)

    def pair_body(p, carry):
        b0 = 2 * p
        pltpu.make_async_copy(idx_hbm.at[bid0 + b0 + 1], i1, semi1).wait()
        issue(i1, sr1, dr1, sem1)
        drain(i0, rc0, sr0, dr0, sem0)
        pltpu.async_copy(idx_hbm.at[bid0 + b0 + 2], i0, semi0)
        compute(b0, rc0, sr0, dr0)
        pltpu.make_async_copy(idx_hbm.at[bid0 + b0 + 2], i0, semi0).wait()
        issue(i0, sr0, dr0, sem0)
        drain(i1, rc1, sr1, dr1, sem1)

        @pl.when(b0 + 3 < NB)
        def _():
            pltpu.async_copy(idx_hbm.at[bid0 + b0 + 3], i1, semi1)

        compute(b0 + 1, rc1, sr1, dr1)
        return carry

    lax.fori_loop(0, NPAIR, pair_body, 0)

    drain(i0, rc0, sr0, dr0, sem0)
    compute(NB - 1, rc0, sr0, dr0)

    pltpu.sync_copy(score, out_hbm.at[pl.ds(wid * EPW, EPW)])


def kernel(h, edge_index, rel_ids, w_relation):
    u = edge_index[0].astype(jnp.int32)
    v = edge_index[1].astype(jnp.int32)
    r = rel_ids.astype(jnp.int32)
    idx3 = (jnp.stack([u, v, r], axis=0)
            .reshape(3, NW * NB, B)
            .transpose(1, 0, 2))
    run = pl.kernel(
        _sc_body,
        mesh=plsc.VectorSubcoreMesh(core_axis_name="c", subcore_axis_name="s"),
        compiler_params=pltpu.CompilerParams(needs_layout_passes=False),
        out_type=jax.ShapeDtypeStruct((N_EDGES,), jnp.float32),
        scratch_types=[
            pltpu.VMEM((3, B), jnp.int32),
            pltpu.VMEM((3, B), jnp.int32),
            pltpu.VMEM((B,), jnp.int32),
            pltpu.VMEM((B,), jnp.int32),
            pltpu.VMEM((B, H_DIM), jnp.float32),
            pltpu.VMEM((B, H_DIM), jnp.float32),
            pltpu.VMEM((B, H_DIM), jnp.float32),
            pltpu.VMEM((B, H_DIM), jnp.float32),
            pltpu.VMEM((NUM_RELS * H_DIM,), jnp.float32),
            pltpu.VMEM((L * L,), jnp.float32),
            pltpu.VMEM((EPW,), jnp.float32),
            pltpu.SemaphoreType.DMA,
            pltpu.SemaphoreType.DMA,
            pltpu.SemaphoreType.DMA,
            pltpu.SemaphoreType.DMA,
        ],
    )
    return run(idx3, w_relation.reshape(-1).astype(jnp.float32),
               h.astype(jnp.float32))
